# Initial kernel scaffold; baseline (speedup 1.0000x reference)
import jax
import jax.numpy as jnp
from jax.experimental import pallas as pl


def kernel(x, edge_index, W, b):
    raise NotImplementedError("write your pallas kernel here")


# speedup vs baseline: 3.5911x; 3.5911x over previous
import functools

import jax
import jax.numpy as jnp
from jax import lax
from jax.experimental import pallas as pl
from jax.experimental.pallas import tpu as pltpu
from jax.experimental.pallas import tpu_sc as plsc

N = 10000
E = 320000
F = 128
K = 7500
TRASH = 2 * N
TBL_A = 2 * N + 8

_NC, _NS = 2, 16
_CH = 80
_IB = 5
_ZR = 1000

_f32 = jnp.float32
_i32 = jnp.int32


def _build_prop(tbl_rows, feat, per_core_gidx):
    etile = E // _NS if per_core_gidx else E // (_NC * _NS)
    nchunks = etile // _CH
    mesh = plsc.VectorSubcoreMesh(core_axis_name="c", subcore_axis_name="s")

    @functools.partial(
        pl.kernel,
        out_type=jax.ShapeDtypeStruct((_NC, N, feat), _f32),
        mesh=mesh,
        scratch_types=[
            pltpu.VMEM((etile,), _i32),
            pltpu.VMEM((etile,), _i32),
            pltpu.VMEM((_CH, feat), _f32),
            pltpu.VMEM_SHARED((N, feat), _f32),
            pltpu.SemaphoreType.DMA,
        ],
        compiler_params=pltpu.CompilerParams(use_tc_tiling_on_sc=False),
    )
    def prop(table, gidx, scol, zeros, out, gbuf, cbuf, rows, acc, sem):
        c = lax.axis_index("c")
        s = lax.axis_index("s")

        @pl.when(s < N // _ZR)
        def _init():
            pltpu.sync_copy(zeros, acc.at[pl.ds(s * _ZR, _ZR)])

        if per_core_gidx:
            base = s * etile
            pltpu.sync_copy(gidx.at[pl.ds(c * E + base, etile)], gbuf)
        else:
            base = (s * _NC + c) * etile
            pltpu.sync_copy(gidx.at[pl.ds(base, etile)], gbuf)
        pltpu.sync_copy(scol.at[pl.ds(base, etile)], cbuf)
        plsc.subcore_barrier()

        @pl.loop(0, nchunks)
        def _chunk(j):
            e0 = j * _CH
            pltpu.async_copy(table.at[gbuf.at[pl.ds(e0, _CH)]],
                             rows, sem).wait()
            pltpu.sync_copy(rows, acc.at[cbuf.at[pl.ds(e0, _CH)]], add=True)

        plsc.subcore_barrier()

        @pl.when(s < N // _ZR)
        def _dump():
            pltpu.sync_copy(acc.at[pl.ds(s * _ZR, _ZR)],
                            out.at[c, pl.ds(s * _ZR, _ZR)])

    return prop


@functools.lru_cache(maxsize=None)
def _get_props():
    return (_build_prop(8, 8, per_core_gidx=False),
            _build_prop(TBL_A, F, per_core_gidx=True),
            _build_prop(N, F, per_core_gidx=False))


def _k0_body(row_ref, col_ref, g_ref, ew_ref):
    r = row_ref[...]
    cc = col_ref[...]
    self_e = r == cc
    g_ref[0] = r
    g_ref[1] = jnp.where(self_e, TRASH, r + N)
    ew_ref[...] = jnp.where(self_e, 0, 1).astype(_i32)


def _kb_body(x_ref, degp_ref, table_ref, xtan_ref, dis_ref):
    x = x_ref[...]
    norm = jnp.sqrt(jnp.sum(x * x, axis=1, keepdims=True))
    norm = jnp.maximum(norm, 1e-15)
    z = jnp.clip(norm, -1.0 + 1e-7, 1.0 - 1e-7)
    artanh = 0.5 * jnp.log((1.0 + z) / (1.0 - z))
    xt = x / norm * artanh
    deg = (degp_ref[0] + degp_ref[1])[:, 0:1]
    dis = jnp.where(deg > 0, lax.rsqrt(jnp.maximum(deg, 1.0)), 0.0)
    table_ref[pl.ds(0, N), :] = xt
    table_ref[pl.ds(N, N), :] = dis * xt
    table_ref[pl.ds(2 * N, 8), :] = jnp.zeros((8, F), _f32)
    xtan_ref[...] = xt
    dis_ref[...] = dis


def _kd_body(a_ref, dis_ref, xtan_ref, z_ref, sel_ref):
    xt = xtan_ref[...]
    info = xt - dis_ref[...] * a_ref[1]
    score = jnp.sum(jnp.abs(info), axis=1, keepdims=True)
    sbits = lax.bitcast_convert_type(score, _i32)

    def step(j, u):
        t = u | (jnp.int32(1) << (30 - j))
        cnt = jnp.sum(jnp.where(sbits >= t, 1.0, 0.0))
        return jnp.where(cnt >= K, t, u)

    u = lax.fori_loop(0, 31, step, jnp.int32(0))
    sel = jnp.where(sbits > u, 1.0, 0.0)
    z_ref[...] = sel * xt
    sel_ref[...] = sel


def _kf_body(c_ref, a_ref, sel_ref, xtan_ref, w_ref, b_ref, u_ref):
    ssel = c_ref[0] + c_ref[1]
    sneigh = a_ref[0]
    logits = (jnp.sum(ssel * w_ref[:, 0:F], axis=1, keepdims=True)
              + jnp.sum(sneigh * w_ref[:, F:2 * F], axis=1, keepdims=True)
              + b_ref[0, 0])
    w = 1.0 / (1.0 + jnp.exp(-logits))
    u_ref[...] = w * sel_ref[...] * xtan_ref[...]


def _kh_body(e_ref, xtan_ref, out_ref):
    a = jnp.maximum(e_ref[0] + e_ref[1], 0.0)
    t = xtan_ref[...] + a
    norm = jnp.maximum(jnp.sqrt(jnp.sum(t * t, axis=1, keepdims=True)), 1e-15)
    e = jnp.tanh(norm) * t / norm
    n2 = jnp.maximum(jnp.sqrt(jnp.sum(e * e, axis=1, keepdims=True)), 1e-15)
    maxn = 1.0 - 1e-5
    out_ref[...] = jnp.where(n2 > maxn, e / n2 * maxn, e)


def _sds(shape, dtype=_f32):
    return jax.ShapeDtypeStruct(shape, dtype)


def kernel(x, edge_index, W, b):
    _prop_deg, _prop_a, _prop_p = _get_props()
    ei = edge_index.astype(_i32)
    row = ei[0]
    col = ei[1]
    row2 = row.reshape(E // F, F)
    col2 = col.reshape(E // F, F)

    g, ew = pl.pallas_call(
        _k0_body,
        out_shape=[_sds((2, E // F, F), _i32), _sds((E // F, F), _i32)],
    )(row2, col2)

    gAr = g.reshape(_NC * E)
    ewr = ew.reshape(E)
    rowr = row
    colr = col
    z8 = jnp.zeros((_ZR, 8), _f32)
    z128 = jnp.zeros((_ZR, F), _f32)
    table8 = jnp.concatenate(
        [jnp.zeros((1, 8), _f32), jnp.ones((1, 8), _f32),
         jnp.zeros((6, 8), _f32)])

    degp = _prop_deg(table8, ewr, rowr, z8)

    table, xtan, dis = pl.pallas_call(
        _kb_body,
        out_shape=[_sds((TBL_A, F)), _sds((N, F)), _sds((N, 1))],
    )(x, degp)

    A = _prop_a(table, gAr, colr, z128)

    z, sel = pl.pallas_call(
        _kd_body,
        out_shape=[_sds((N, F)), _sds((N, 1))],
    )(A, dis, xtan)

    C = _prop_p(z, rowr, colr, z128)

    u = pl.pallas_call(
        _kf_body,
        out_shape=_sds((N, F)),
    )(C, A, sel, xtan, W, b.reshape(1, 1))

    Eo = _prop_p(u, rowr, colr, z128)

    out = pl.pallas_call(
        _kh_body,
        out_shape=_sds((N, F)),
    )(Eo, xtan)
    return out
